# baseline (speedup 1.0000x reference)
import jax
import jax.numpy as jnp
from jax import lax
from jax.experimental import pallas as pl
from jax.experimental.pallas import tpu as pltpu
from jax.experimental.pallas import tpu_sc as plsc

DIM = 4096
BATCH = 16384
ROWS = 4

_INFO = plsc.get_sparse_core_info()
_NC = _INFO.num_cores
_NS = _INFO.num_subcores
_NW = _NC * _NS
_BPW = BATCH // _NW
_NTRI = 96
_NPAIR = 112
_G = 16
_LAG = 2
_SCHED = ["T", "P"] * 6 + ["P"]


def _body(ids_hbm, table_hbm, out_hbm, ids_v, trip_s, bsem, dsem, psem):
    cid = lax.axis_index("c")
    sid = lax.axis_index("s")
    wid = sid * _NC + cid
    base = wid * _BPW
    pltpu.sync_copy(ids_hbm.at[pl.ds(base, _BPW)], ids_v)

    for k in range(4):
        c = sid * 4 + k
        r0 = c // 16
        r1 = (c // 4) % 4
        r2 = c % 4
        pltpu.async_copy(table_hbm.at[pl.ds(r0 * DIM, DIM)], trip_s.at[pl.ds(c * 3 * DIM, DIM)], bsem)
        pltpu.async_copy(table_hbm.at[pl.ds(r1 * DIM, DIM)], trip_s.at[pl.ds((c * 3 + 1) * DIM, DIM)], bsem)
        pltpu.async_copy(table_hbm.at[pl.ds(r2 * DIM, DIM)], trip_s.at[pl.ds((c * 3 + 2) * DIM, DIM)], bsem)
    for _ in range(12):
        pltpu.make_async_copy(table_hbm.at[pl.ds(0, DIM)], trip_s.at[pl.ds(0, DIM)], bsem).wait()
    plsc.subcore_barrier()

    def issue_triples(tb):
        i0 = 3 * tb
        vs = (
            ids_v[pl.ds(i0, 16)],
            ids_v[pl.ds(i0 + 16, 16)],
            ids_v[pl.ds(i0 + 32, 16)],
        )
        for j in range(_G):
            e0 = vs[(3 * j) // 16][(3 * j) % 16]
            e1 = vs[(3 * j + 1) // 16][(3 * j + 1) % 16]
            e2 = vs[(3 * j + 2) // 16][(3 * j + 2) % 16]
            combo = e0 * 16 + e1 * 4 + e2
            pltpu.async_copy(
                trip_s.at[pl.ds(combo * (3 * DIM), 3 * DIM)],
                out_hbm.at[pl.ds((base + 3 * (tb + j)) * DIM, 3 * DIM)],
                dsem,
            )

    def issue_pairs(pb):
        i0 = 3 * _NTRI + 2 * pb
        ws = (
            ids_v[pl.ds(i0, 16)],
            ids_v[pl.ds(i0 + 16, 16)],
        )
        for j in range(_G):
            f0 = ws[(2 * j) // 16][(2 * j) % 16]
            f1 = ws[(2 * j + 1) // 16][(2 * j + 1) % 16]
            combo = f0 * 16 + f1 * 4
            pltpu.async_copy(
                trip_s.at[pl.ds(combo * (3 * DIM), 2 * DIM)],
                out_hbm.at[pl.ds((base + 3 * _NTRI + 2 * (pb + j)) * DIM, 2 * DIM)],
                psem,
            )

    def drain_triples():
        for _ in range(_G):
            pltpu.make_async_copy(trip_s.at[pl.ds(0, 3 * DIM)], out_hbm.at[pl.ds(0, 3 * DIM)], dsem).wait()

    def drain_pairs():
        for _ in range(_G):
            pltpu.make_async_copy(trip_s.at[pl.ds(0, 2 * DIM)], out_hbm.at[pl.ds(0, 2 * DIM)], psem).wait()

    t_issued = 0
    p_issued = 0
    t_drained = 0
    p_drained = 0
    for typ in _SCHED:
        if typ == "T":
            if t_issued - t_drained >= _LAG:
                drain_triples()
                t_drained += 1
            issue_triples(t_issued * _G)
            t_issued += 1
        else:
            if p_issued - p_drained >= _LAG:
                drain_pairs()
                p_drained += 1
            issue_pairs(p_issued * _G)
            p_issued += 1
    while t_drained < t_issued:
        drain_triples()
        t_drained += 1
    while p_drained < p_issued:
        drain_pairs()
        p_drained += 1


def kernel(step_ids, step_embeddings):
    ids = step_ids.astype(jnp.int32)
    out = pl.kernel(
        _body,
        out_type=jax.ShapeDtypeStruct((BATCH * DIM,), jnp.float32),
        mesh=plsc.VectorSubcoreMesh(core_axis_name="c", subcore_axis_name="s"),
        scratch_types=[
            pltpu.VMEM((_BPW,), jnp.int32),
            pltpu.VMEM_SHARED((64 * 3 * DIM,), jnp.float32),
            pltpu.SemaphoreType.DMA,
            pltpu.SemaphoreType.DMA,
            pltpu.SemaphoreType.DMA,
        ],
    )(ids, step_embeddings.reshape(-1))
    return out.reshape(BATCH, 1, DIM)


# speedup vs baseline: 1.3185x; 1.3185x over previous
import jax
import jax.numpy as jnp
from jax import lax
from jax.experimental import pallas as pl
from jax.experimental.pallas import tpu as pltpu
from jax.experimental.pallas import tpu_sc as plsc

DIM = 4096
BATCH = 16384
ROWS = 4

_INFO = plsc.get_sparse_core_info()
_NC = _INFO.num_cores
_NS = _INFO.num_subcores
_NW = _NC * _NS
_BPW = BATCH // _NW
_NTRI = 96
_NPAIR = 112
_TG = 16
_PG = 16
_LAG = 2
_SCHED = ["T", "P"] * 6 + ["P"]


def _de_bruijn(k, n):
    a = [0] * (k * n)
    seq = []

    def db(t, p):
        if t > n:
            if n % p == 0:
                seq.extend(a[1:p + 1])
        else:
            a[t] = a[t - p]
            db(t + 1, p)
            for j in range(a[t - p] + 1, k):
                a[t] = j
                db(t + 1, t)

    db(1, 1)
    return seq


_DB3 = _de_bruijn(4, 3)
_DB3 = _DB3 + _DB3[:2]
_POS3 = [None] * 64
for _i in range(64):
    _c = _DB3[_i] * 16 + _DB3[_i + 1] * 4 + _DB3[_i + 2]
    if _POS3[_c] is None:
        _POS3[_c] = _i
_DB2 = _de_bruijn(4, 2)
_DB2 = _DB2 + _DB2[:1]
_POS2 = [None] * 16
for _i in range(16):
    _c = _DB2[_i] * 4 + _DB2[_i + 1]
    if _POS2[_c] is None:
        _POS2[_c] = _i

_NDB3 = len(_DB3)
_NDB2 = len(_DB2)


def _body(ids_hbm, table_hbm, out_hbm, ids_v, pair_v, trip_s, pos3_m, pos2_m,
          bsem, csem, qsem, psem):
    cid = lax.axis_index("c")
    sid = lax.axis_index("s")
    wid = sid * _NC + cid
    base = wid * _BPW
    pltpu.sync_copy(ids_hbm.at[pl.ds(base, _BPW)], ids_v)

    for r in range(_NDB2):
        pltpu.async_copy(table_hbm.at[pl.ds(_DB2[r] * DIM, DIM)],
                         pair_v.at[pl.ds(r * DIM, DIM)], csem)

    for r in range(_NDB3):
        @pl.when(sid == r % _NS)
        def _():
            pltpu.async_copy(table_hbm.at[pl.ds(_DB3[r] * DIM, DIM)],
                             trip_s.at[pl.ds(r * DIM, DIM)], bsem)

    for c in range(64):
        pos3_m[c] = _POS3[c]
    for c in range(16):
        pos2_m[c] = _POS2[c]

    for _ in range(_NDB2):
        pltpu.make_async_copy(table_hbm.at[pl.ds(0, DIM)], pair_v.at[pl.ds(0, DIM)], csem).wait()
    for _ in range(_NDB3 // _NS):
        pltpu.make_async_copy(table_hbm.at[pl.ds(0, DIM)], trip_s.at[pl.ds(0, DIM)], bsem).wait()

    @pl.when(sid < _NDB3 % _NS)
    def _():
        pltpu.make_async_copy(table_hbm.at[pl.ds(0, DIM)], trip_s.at[pl.ds(0, DIM)], bsem).wait()

    plsc.subcore_barrier()

    def issue_triples(tb):
        i0 = 3 * tb
        vs = (
            ids_v[pl.ds(i0, 16)],
            ids_v[pl.ds(i0 + 16, 16)],
            ids_v[pl.ds(i0 + 32, 16)],
        )
        for j in range(_TG):
            e0 = vs[(3 * j) // 16][(3 * j) % 16]
            e1 = vs[(3 * j + 1) // 16][(3 * j + 1) % 16]
            e2 = vs[(3 * j + 2) // 16][(3 * j + 2) % 16]
            pos = pos3_m[e0 * 16 + e1 * 4 + e2]
            pltpu.async_copy(
                trip_s.at[pl.ds(pos * DIM, 3 * DIM)],
                out_hbm.at[pl.ds((base + 3 * (tb + j)) * DIM, 3 * DIM)],
                qsem,
            )

    def issue_pairs(pb):
        i0 = 3 * _NTRI + 2 * pb
        ws = (
            ids_v[pl.ds(i0, 16)],
            ids_v[pl.ds(i0 + 16, 16)],
        )
        for j in range(_PG):
            f0 = ws[(2 * j) // 16][(2 * j) % 16]
            f1 = ws[(2 * j + 1) // 16][(2 * j + 1) % 16]
            pos = pos2_m[f0 * 4 + f1]
            pltpu.async_copy(
                pair_v.at[pl.ds(pos * DIM, 2 * DIM)],
                out_hbm.at[pl.ds((base + 3 * _NTRI + 2 * (pb + j)) * DIM, 2 * DIM)],
                psem,
            )

    def drain_triples():
        for _ in range(_TG):
            pltpu.make_async_copy(trip_s.at[pl.ds(0, 3 * DIM)], out_hbm.at[pl.ds(0, 3 * DIM)], qsem).wait()

    def drain_pairs():
        for _ in range(_PG):
            pltpu.make_async_copy(pair_v.at[pl.ds(0, 2 * DIM)], out_hbm.at[pl.ds(0, 2 * DIM)], psem).wait()

    q_issued = 0
    p_issued = 0
    q_drained = 0
    p_drained = 0
    for typ in _SCHED:
        if typ == "T":
            if q_issued - q_drained >= _LAG:
                drain_triples()
                q_drained += 1
            issue_triples(q_issued * _TG)
            q_issued += 1
        else:
            if p_issued - p_drained >= _LAG:
                drain_pairs()
                p_drained += 1
            issue_pairs(p_issued * _PG)
            p_issued += 1
    while q_drained < q_issued:
        drain_triples()
        q_drained += 1
    while p_drained < p_issued:
        drain_pairs()
        p_drained += 1


def kernel(step_ids, step_embeddings):
    ids = step_ids.astype(jnp.int32)
    out = pl.kernel(
        _body,
        out_type=jax.ShapeDtypeStruct((BATCH * DIM,), jnp.float32),
        mesh=plsc.VectorSubcoreMesh(core_axis_name="c", subcore_axis_name="s"),
        scratch_types=[
            pltpu.VMEM((_BPW,), jnp.int32),
            pltpu.VMEM((_NDB2 * DIM,), jnp.float32),
            pltpu.VMEM_SHARED((_NDB3 * DIM,), jnp.float32),
            pltpu.SMEM((64,), jnp.int32),
            pltpu.SMEM((16,), jnp.int32),
            pltpu.SemaphoreType.DMA,
            pltpu.SemaphoreType.DMA,
            pltpu.SemaphoreType.DMA,
            pltpu.SemaphoreType.DMA,
        ],
    )(ids, step_embeddings.reshape(-1))
    return out.reshape(BATCH, 1, DIM)


# speedup vs baseline: 1.3207x; 1.0016x over previous
import jax
import jax.numpy as jnp
from jax import lax
from jax.experimental import pallas as pl
from jax.experimental.pallas import tpu as pltpu
from jax.experimental.pallas import tpu_sc as plsc

DIM = 4096
BATCH = 16384
ROWS = 4

_INFO = plsc.get_sparse_core_info()
_NC = _INFO.num_cores
_NS = _INFO.num_subcores
_NW = _NC * _NS
_BPW = BATCH // _NW
_NTRI = 96
_NPAIR = 112
_TG = 16
_PG = 16
_LAG = 4
_SCHED = ["T", "P"] * 6 + ["P"]


def _de_bruijn(k, n):
    a = [0] * (k * n)
    seq = []

    def db(t, p):
        if t > n:
            if n % p == 0:
                seq.extend(a[1:p + 1])
        else:
            a[t] = a[t - p]
            db(t + 1, p)
            for j in range(a[t - p] + 1, k):
                a[t] = j
                db(t + 1, t)

    db(1, 1)
    return seq


_DB3 = _de_bruijn(4, 3)
_DB3 = _DB3 + _DB3[:2]
_POS3 = [None] * 64
for _i in range(64):
    _c = _DB3[_i] * 16 + _DB3[_i + 1] * 4 + _DB3[_i + 2]
    if _POS3[_c] is None:
        _POS3[_c] = _i
_DB2 = _de_bruijn(4, 2)
_DB2 = _DB2 + _DB2[:1]
_POS2 = [None] * 16
for _i in range(16):
    _c = _DB2[_i] * 4 + _DB2[_i + 1]
    if _POS2[_c] is None:
        _POS2[_c] = _i

_NDB3 = len(_DB3)
_NDB2 = len(_DB2)


def _body(ids_hbm, table_hbm, out_hbm, ids_v, pair_v, trip_s, pos3_m, pos2_m,
          bsem, csem, qsem, psem):
    cid = lax.axis_index("c")
    sid = lax.axis_index("s")
    wid = sid * _NC + cid
    base = wid * _BPW
    pltpu.sync_copy(ids_hbm.at[pl.ds(base, _BPW)], ids_v)

    for r in range(_NDB2):
        pltpu.async_copy(table_hbm.at[pl.ds(_DB2[r] * DIM, DIM)],
                         pair_v.at[pl.ds(r * DIM, DIM)], csem)

    for r in range(_NDB3):
        @pl.when(sid == r % _NS)
        def _():
            pltpu.async_copy(table_hbm.at[pl.ds(_DB3[r] * DIM, DIM)],
                             trip_s.at[pl.ds(r * DIM, DIM)], bsem)

    for c in range(64):
        pos3_m[c] = _POS3[c]
    for c in range(16):
        pos2_m[c] = _POS2[c]

    for _ in range(_NDB2):
        pltpu.make_async_copy(table_hbm.at[pl.ds(0, DIM)], pair_v.at[pl.ds(0, DIM)], csem).wait()
    for _ in range(_NDB3 // _NS):
        pltpu.make_async_copy(table_hbm.at[pl.ds(0, DIM)], trip_s.at[pl.ds(0, DIM)], bsem).wait()

    @pl.when(sid < _NDB3 % _NS)
    def _():
        pltpu.make_async_copy(table_hbm.at[pl.ds(0, DIM)], trip_s.at[pl.ds(0, DIM)], bsem).wait()

    plsc.subcore_barrier()

    def issue_triples(tb):
        i0 = 3 * tb
        vs = (
            ids_v[pl.ds(i0, 16)],
            ids_v[pl.ds(i0 + 16, 16)],
            ids_v[pl.ds(i0 + 32, 16)],
        )
        for j in range(_TG):
            e0 = vs[(3 * j) // 16][(3 * j) % 16]
            e1 = vs[(3 * j + 1) // 16][(3 * j + 1) % 16]
            e2 = vs[(3 * j + 2) // 16][(3 * j + 2) % 16]
            pos = pos3_m[e0 * 16 + e1 * 4 + e2]
            pltpu.async_copy(
                trip_s.at[pl.ds(pos * DIM, 3 * DIM)],
                out_hbm.at[pl.ds((base + 3 * (tb + j)) * DIM, 3 * DIM)],
                qsem,
            )

    def issue_pairs(pb):
        i0 = 3 * _NTRI + 2 * pb
        ws = (
            ids_v[pl.ds(i0, 16)],
            ids_v[pl.ds(i0 + 16, 16)],
        )
        for j in range(_PG):
            f0 = ws[(2 * j) // 16][(2 * j) % 16]
            f1 = ws[(2 * j + 1) // 16][(2 * j + 1) % 16]
            pos = pos2_m[f0 * 4 + f1]
            pltpu.async_copy(
                pair_v.at[pl.ds(pos * DIM, 2 * DIM)],
                out_hbm.at[pl.ds((base + 3 * _NTRI + 2 * (pb + j)) * DIM, 2 * DIM)],
                psem,
            )

    def drain_triples():
        for _ in range(_TG):
            pltpu.make_async_copy(trip_s.at[pl.ds(0, 3 * DIM)], out_hbm.at[pl.ds(0, 3 * DIM)], qsem).wait()

    def drain_pairs():
        for _ in range(_PG):
            pltpu.make_async_copy(pair_v.at[pl.ds(0, 2 * DIM)], out_hbm.at[pl.ds(0, 2 * DIM)], psem).wait()

    q_issued = 0
    p_issued = 0
    q_drained = 0
    p_drained = 0
    for typ in _SCHED:
        if typ == "T":
            if q_issued - q_drained >= _LAG:
                drain_triples()
                q_drained += 1
            issue_triples(q_issued * _TG)
            q_issued += 1
        else:
            if p_issued - p_drained >= _LAG:
                drain_pairs()
                p_drained += 1
            issue_pairs(p_issued * _PG)
            p_issued += 1
    while q_drained < q_issued:
        drain_triples()
        q_drained += 1
    while p_drained < p_issued:
        drain_pairs()
        p_drained += 1


def kernel(step_ids, step_embeddings):
    ids = step_ids.astype(jnp.int32)
    out = pl.kernel(
        _body,
        out_type=jax.ShapeDtypeStruct((BATCH * DIM,), jnp.float32),
        mesh=plsc.VectorSubcoreMesh(core_axis_name="c", subcore_axis_name="s"),
        scratch_types=[
            pltpu.VMEM((_BPW,), jnp.int32),
            pltpu.VMEM((_NDB2 * DIM,), jnp.float32),
            pltpu.VMEM_SHARED((_NDB3 * DIM,), jnp.float32),
            pltpu.SMEM((64,), jnp.int32),
            pltpu.SMEM((16,), jnp.int32),
            pltpu.SemaphoreType.DMA,
            pltpu.SemaphoreType.DMA,
            pltpu.SemaphoreType.DMA,
            pltpu.SemaphoreType.DMA,
        ],
    )(ids, step_embeddings.reshape(-1))
    return out.reshape(BATCH, 1, DIM)
